# baseline (speedup 1.0000x reference)
import functools

import jax
import jax.numpy as jnp
from jax import lax
from jax.experimental import pallas as pl
from jax.experimental.pallas import tpu as pltpu
from jax.experimental.pallas import tpu_sc as plsc

D_MODEL = 1024
VOCAB = 60
VPAD = 64
LANES = 16
BLK = 1024

NW = 32
DEPTH = 16
SC_B = 1
TC_B = 4 - SC_B
SC_ROWS = SC_B * 4096
PER_W = SC_ROWS // NW


def _build_sc():
  mesh = plsc.VectorSubcoreMesh(core_axis_name="c", subcore_axis_name="s")

  @functools.partial(
      pl.kernel,
      mesh=mesh,
      out_type=jax.ShapeDtypeStruct((SC_B, 4096, 1, D_MODEL), jnp.float32),
      scratch_types=[
          pltpu.VMEM((VOCAB, D_MODEL), jnp.float32),
          pltpu.VMEM((PER_W + LANES,), jnp.int32),
          pltpu.SemaphoreType.DMA,
          pltpu.SemaphoreType.DMA,
      ],
  )
  def emb(table_hbm, idx_hbm, out_hbm, tab_v, idx_v, sem, sem_in):
    wid = lax.axis_index("s") * 2 + lax.axis_index("c")
    cp_tab = pltpu.async_copy(table_hbm, tab_v, sem_in)
    cp_idx = pltpu.async_copy(idx_hbm.at[pl.ds(wid * PER_W, PER_W)],
                              idx_v.at[pl.ds(0, PER_W)], sem_in)
    cp_tab.wait()
    cp_idx.wait()

    row0 = wid * PER_W
    t0 = row0 % 4096

    def fire(j):
      iv = idx_v[pl.ds(j, LANES)]
      v = iv[0]
      pltpu.async_copy(
          tab_v.at[pl.ds(v, 1)],
          out_hbm.at[row0 // 4096, pl.ds(t0 + j, 1), 0],
          sem)

    def wait_one():
      pltpu.make_async_copy(
          tab_v.at[pl.ds(0, 1)],
          out_hbm.at[0, pl.ds(0, 1), 0],
          sem).wait()

    def prime(j, _):
      fire(j)
      return 0

    lax.fori_loop(0, DEPTH, prime, 0)

    def body(j, _):
      wait_one()
      fire(j + DEPTH)
      return 0

    lax.fori_loop(0, PER_W - DEPTH, body, 0)

    def drain(j, _):
      wait_one()
      return 0

    lax.fori_loop(0, DEPTH, drain, 0)

  return emb


_sc_gather = _build_sc()


def _tc_body(idx_ref, w_ref, o_ref):
  idx = idx_ref[0, 0, :]
  iota = jax.lax.broadcasted_iota(jnp.int32, (BLK, VPAD), 1)
  onehot = (idx[:, None] == iota).astype(jnp.bfloat16)
  o_ref[0, :, 0, :] = jnp.dot(onehot, w_ref[...].astype(jnp.bfloat16),
                              preferred_element_type=jnp.float32)


def _tc_gather(idx3, w_pad):
  return pl.pallas_call(
      _tc_body,
      grid=(TC_B * 4096 // BLK,),
      in_specs=[
          pl.BlockSpec((1, 1, BLK), lambda i: (i, 0, 0)),
          pl.BlockSpec((VPAD, D_MODEL), lambda i: (0, 0)),
      ],
      out_specs=pl.BlockSpec((1, BLK, 1, D_MODEL),
                             lambda i: (i // 4, i % 4, 0, 0)),
      out_shape=jax.ShapeDtypeStruct((TC_B, 4096, 1, D_MODEL), jnp.float32),
  )(idx3, w_pad)


def kernel(x_mark, W):
  idx = x_mark[:, :, 1].astype(jnp.int32)
  idx_tc = idx[:TC_B].reshape(TC_B * 4096 // BLK, 1, BLK)
  idx_sc = idx[TC_B:].reshape(SC_ROWS)
  w_pad = jnp.pad(W, ((0, VPAD - VOCAB), (0, 0)))
  out_sc = _sc_gather(W, idx_sc)
  out_tc = _tc_gather(idx_tc, w_pad)
  return jnp.concatenate([out_tc, out_sc], axis=0)


# speedup vs baseline: 2.3693x; 2.3693x over previous
import functools

import jax
import jax.numpy as jnp
from jax import lax
from jax.experimental import pallas as pl
from jax.experimental.pallas import tpu as pltpu
from jax.experimental.pallas import tpu_sc as plsc

D_MODEL = 1024
VOCAB = 60
LANES = 16
NW = 32
DEPTH = 16
PER_W = 512
ROWS = NW * PER_W


def _build():
  mesh = plsc.VectorSubcoreMesh(core_axis_name="c", subcore_axis_name="s")

  @functools.partial(
      pl.kernel,
      mesh=mesh,
      out_type=jax.ShapeDtypeStruct((4, ROWS // 4, 1, D_MODEL), jnp.float32),
      scratch_types=[
          pltpu.VMEM((VOCAB, D_MODEL), jnp.float32),
          pltpu.VMEM((PER_W + LANES,), jnp.int32),
          pltpu.SemaphoreType.DMA,
          pltpu.SemaphoreType.DMA,
      ],
  )
  def emb(table_hbm, idx_hbm, out_hbm, tab_v, idx_v, sem, sem_in):
    wid = lax.axis_index("s") * 2 + lax.axis_index("c")
    cp_tab = pltpu.async_copy(table_hbm, tab_v, sem_in)
    cp_idx = pltpu.async_copy(idx_hbm.at[pl.ds(wid * PER_W, PER_W)],
                              idx_v.at[pl.ds(0, PER_W)], sem_in)
    cp_tab.wait()
    cp_idx.wait()

    row0 = wid * PER_W
    bidx = row0 // 4096
    t0 = row0 % 4096

    def fire(j):
      iv = idx_v[pl.ds(j, LANES)]
      v = iv[0]
      pltpu.async_copy(
          tab_v.at[pl.ds(v, 1)],
          out_hbm.at[bidx, pl.ds(t0 + j, 1), 0],
          sem)

    def wait_one():
      pltpu.make_async_copy(
          tab_v.at[pl.ds(0, 1)],
          out_hbm.at[bidx, pl.ds(t0, 1), 0],
          sem).wait()

    def prime(j, _):
      fire(j)
      return 0

    lax.fori_loop(0, DEPTH, prime, 0)

    def body(j, _):
      wait_one()
      fire(j + DEPTH)
      return 0

    lax.fori_loop(0, PER_W - DEPTH, body, 0)

    def drain(j, _):
      wait_one()
      return 0

    lax.fori_loop(0, DEPTH, drain, 0)

  return emb


_emb = _build()


def kernel(x_mark, W):
  idx = x_mark[:, :, 1].reshape(ROWS).astype(jnp.int32)
  return _emb(W, idx)
